# baseline (speedup 1.0000x reference)
import functools

import jax
import jax.numpy as jnp
from jax import lax
from jax.experimental import pallas as pl
from jax.experimental.pallas import tpu as pltpu
from jax.experimental.pallas import tpu_sc as plsc

_NC = 2
_NS = 16
_NW = _NC * _NS
_L = 16

_CH = 16


def _sc_add_row(x_hbm, tbl_hbm, idx_hbm, out_hbm, idx_v, row_v, buf_v, sem):
    D = 1024
    rows_per_w = 32768 // _NW
    n_chunks = rows_per_w // _CH

    wid = lax.axis_index("s") * _NC + lax.axis_index("c")
    base = wid * rows_per_w

    pltpu.sync_copy(idx_hbm, idx_v)
    pltpu.async_copy(tbl_hbm.at[idx_v], row_v, sem).wait()

    def chunk_body(ci, _):
        r0 = base + ci * _CH
        pltpu.sync_copy(x_hbm.at[pl.ds(r0, _CH)], buf_v)

        def add_body(k, _):
            i = k // (D // _L)
            j = (k % (D // _L)) * _L
            buf_v[i, pl.ds(j, _L)] = buf_v[i, pl.ds(j, _L)] + row_v[i, pl.ds(j, _L)]
            return 0

        lax.fori_loop(0, _CH * (D // _L), add_body, 0)
        pltpu.sync_copy(buf_v, out_hbm.at[pl.ds(r0, _CH)])
        return 0

    lax.fori_loop(0, n_chunks, chunk_body, 0)


def kernel(x, tf_table, tf_id):
    B, S, D = x.shape
    R = B * S
    xf = x.reshape(R, D)
    idx = jnp.full((_L,), tf_id, dtype=jnp.int32)

    mesh = plsc.VectorSubcoreMesh(core_axis_name="c", subcore_axis_name="s")
    run = functools.partial(
        pl.kernel,
        mesh=mesh,
        out_type=jax.ShapeDtypeStruct((R, D), x.dtype),
        scratch_types=[
            pltpu.VMEM((_L,), jnp.int32),
            pltpu.VMEM((_L, D), jnp.float32),
            pltpu.VMEM((_CH, D), jnp.float32),
            pltpu.SemaphoreType.DMA,
        ],
    )(_sc_add_row)
    out = run(xf, tf_table, idx)
    return out.reshape(B, S, D)


# speedup vs baseline: 2.4584x; 2.4584x over previous
import functools

import jax
import jax.numpy as jnp
from jax import lax
from jax.experimental import pallas as pl
from jax.experimental.pallas import tpu as pltpu
from jax.experimental.pallas import tpu_sc as plsc

_NC = 2
_NS = 16
_NW = _NC * _NS
_L = 16

_D = 1024
_R = 32768
_CH = 16
_RPW = _R // _NW
_NCHUNK = _RPW // _CH


def _sc_add_row(x_hbm, tbl_hbm, idx_hbm, out_hbm,
                idx_v, row_v, in0, in1, ot0, ot1,
                gsem, isem0, isem1, osem0, osem1):
    wid = lax.axis_index("s") * _NC + lax.axis_index("c")
    base = wid * _RPW

    pltpu.sync_copy(idx_hbm, idx_v)
    pltpu.async_copy(tbl_hbm.at[idx_v], row_v, gsem).wait()

    def start_in(c, buf, sem):
        pltpu.async_copy(x_hbm.at[pl.ds(base + c * _CH, _CH)], buf, sem)

    def wait_in(c, buf, sem):
        pltpu.make_async_copy(x_hbm.at[pl.ds(base + c * _CH, _CH)], buf, sem).wait()

    def start_out(c, buf, sem):
        pltpu.async_copy(buf, out_hbm.at[pl.ds(base + c * _CH, _CH)], sem)

    def wait_out(c, buf, sem):
        pltpu.make_async_copy(buf, out_hbm.at[pl.ds(base + c * _CH, _CH)], sem).wait()

    def compute(inb, outb):
        def jbody(j, _):
            col = j * _L
            rv = row_v[0, pl.ds(col, _L)]
            for i in range(_CH):
                outb[i, pl.ds(col, _L)] = inb[i, pl.ds(col, _L)] + rv
            return 0
        lax.fori_loop(0, _D // _L, jbody, 0, unroll=2)

    start_in(0, in0, isem0)
    start_in(1, in1, isem1)

    wait_in(0, in0, isem0)
    compute(in0, ot0)
    start_in(2, in0, isem0)
    start_out(0, ot0, osem0)

    wait_in(1, in1, isem1)
    compute(in1, ot1)
    start_in(3, in1, isem1)
    start_out(1, ot1, osem1)

    def gbody(g, _):
        c0 = 2 * g
        wait_out(c0 - 2, ot0, osem0)
        wait_in(c0, in0, isem0)
        compute(in0, ot0)
        start_in(c0 + 2, in0, isem0)
        start_out(c0, ot0, osem0)

        c1 = c0 + 1
        wait_out(c1 - 2, ot1, osem1)
        wait_in(c1, in1, isem1)
        compute(in1, ot1)
        start_in(c1 + 2, in1, isem1)
        start_out(c1, ot1, osem1)
        return 0

    lax.fori_loop(1, _NCHUNK // 2 - 1, gbody, 0)

    cl0 = _NCHUNK - 2
    wait_out(cl0 - 2, ot0, osem0)
    wait_in(cl0, in0, isem0)
    compute(in0, ot0)
    start_out(cl0, ot0, osem0)

    cl1 = _NCHUNK - 1
    wait_out(cl1 - 2, ot1, osem1)
    wait_in(cl1, in1, isem1)
    compute(in1, ot1)
    start_out(cl1, ot1, osem1)

    wait_out(cl0, ot0, osem0)
    wait_out(cl1, ot1, osem1)


def kernel(x, tf_table, tf_id):
    B, S, D = x.shape
    R = B * S
    xf = x.reshape(R, D)
    idx = jnp.full((_L,), tf_id, dtype=jnp.int32)

    mesh = plsc.VectorSubcoreMesh(core_axis_name="c", subcore_axis_name="s")
    run = functools.partial(
        pl.kernel,
        mesh=mesh,
        out_type=jax.ShapeDtypeStruct((R, D), x.dtype),
        scratch_types=[
            pltpu.VMEM((_L,), jnp.int32),
            pltpu.VMEM((_L, _D), jnp.float32),
            pltpu.VMEM((_CH, _D), jnp.float32),
            pltpu.VMEM((_CH, _D), jnp.float32),
            pltpu.VMEM((_CH, _D), jnp.float32),
            pltpu.VMEM((_CH, _D), jnp.float32),
            pltpu.SemaphoreType.DMA,
            pltpu.SemaphoreType.DMA,
            pltpu.SemaphoreType.DMA,
            pltpu.SemaphoreType.DMA,
            pltpu.SemaphoreType.DMA,
        ],
    )(_sc_add_row)
    out = run(xf, tf_table, idx)
    return out.reshape(B, S, D)
